# Initial kernel scaffold; baseline (speedup 1.0000x reference)
import jax
import jax.numpy as jnp
from jax.experimental import pallas as pl


def kernel(indices, emb):
    raise NotImplementedError("write your pallas kernel here")


# speedup vs baseline: 8.5355x; 8.5355x over previous
import functools

import jax
import jax.numpy as jnp
from jax import lax
from jax.experimental import pallas as pl
from jax.experimental.pallas import tpu as pltpu
from jax.experimental.pallas import tpu_sc as plsc

_K = 8
_CARD = 2051
_D = 128
_B = 16
_T = 2048
_NC = 2
_NS = 16
_NW = _NC * _NS
_ROWS = _B * _T
_RPW = _ROWS // _NW
_CHUNK = 16
_GROWS = _K * _CHUNK
_NCHUNK = _RPW // _CHUNK
_LANES = 16


def _body(emb_hbm, idx_hbm, out_hbm, idxraw, idx2, gbuf, obuf,
          gsem0, gsem1, osem0, osem1):
    wid = lax.axis_index("c") * _NS + lax.axis_index("s")
    b = wid // 2
    half = wid % 2
    base = wid * _RPW

    for k in range(_K):
        pltpu.sync_copy(idx_hbm.at[b * _K + k, pl.ds(half * _RPW, _RPW)],
                        idxraw.at[k])

    def build_idx(c, carry):
        for k in range(_K):
            idx2[c, pl.ds(k * _LANES, _LANES)] = (
                idxraw[k, pl.ds(c * _CHUNK, _CHUNK)] + k * _CARD)
        return carry
    lax.fori_loop(0, _NCHUNK, build_idx, 0)

    gsems = (gsem0, gsem1)
    osems = (osem0, osem1)

    def fire_gather(c, s):
        pltpu.async_copy(emb_hbm.at[idx2.at[c]], gbuf.at[s], gsems[s])

    def drain_gather(s):
        pltpu.make_async_copy(emb_hbm.at[pl.ds(0, _GROWS)], gbuf.at[s],
                              gsems[s]).wait()

    def drain_out(s):
        pltpu.make_async_copy(obuf.at[s], out_hbm.at[pl.ds(base, _CHUNK)],
                              osems[s]).wait()

    for s in range(2):
        fire_gather(s, s)

    def outer(g, carry):
        for s in range(2):
            c = g * 2 + s
            drain_gather(s)

            @pl.when(c >= 2)
            def _():
                drain_out(s)

            def sum_rows(r, rc):
                for col in range(_D // _LANES):
                    ds_ = pl.ds(col * _LANES, _LANES)
                    v0 = gbuf[s, 0 * _CHUNK + r, ds_] + gbuf[s, 1 * _CHUNK + r, ds_]
                    v1 = gbuf[s, 2 * _CHUNK + r, ds_] + gbuf[s, 3 * _CHUNK + r, ds_]
                    v2 = gbuf[s, 4 * _CHUNK + r, ds_] + gbuf[s, 5 * _CHUNK + r, ds_]
                    v3 = gbuf[s, 6 * _CHUNK + r, ds_] + gbuf[s, 7 * _CHUNK + r, ds_]
                    obuf[s, r, ds_] = (v0 + v1) + (v2 + v3)
                return rc
            lax.fori_loop(0, _CHUNK, sum_rows, 0)

            pltpu.async_copy(obuf.at[s],
                             out_hbm.at[pl.ds(base + c * _CHUNK, _CHUNK)],
                             osems[s])

            @pl.when(c + 2 < _NCHUNK)
            def _():
                fire_gather(c + 2, s)
        return carry
    lax.fori_loop(0, _NCHUNK // 2, outer, 0)

    drain_out(0)
    drain_out(1)


@functools.partial(jax.jit, static_argnums=())
def _embed_sum(emb2d, idx2d):
    mesh = plsc.VectorSubcoreMesh(core_axis_name="c", subcore_axis_name="s")
    kfn = pl.kernel(
        _body,
        out_type=jax.ShapeDtypeStruct((_ROWS, _D), jnp.float32),
        mesh=mesh,
        scratch_types=[
            pltpu.VMEM((_K, _RPW), jnp.int32),
            pltpu.VMEM((_NCHUNK, _GROWS), jnp.int32),
            pltpu.VMEM((2, _GROWS, _D), jnp.float32),
            pltpu.VMEM((2, _CHUNK, _D), jnp.float32),
            pltpu.SemaphoreType.DMA,
            pltpu.SemaphoreType.DMA,
            pltpu.SemaphoreType.DMA,
            pltpu.SemaphoreType.DMA,
        ],
    )
    return kfn(emb2d, idx2d)


def kernel(indices, emb):
    idx2d = indices.reshape(_B * _K, _T).astype(jnp.int32)
    emb2d = emb.reshape(_K * _CARD, _D)
    out = _embed_sum(emb2d, idx2d)
    return out.reshape(_B, _T, _D)
